# baseline (speedup 1.0000x reference)
import functools

import jax
import jax.numpy as jnp
from jax import lax
from jax.experimental import pallas as pl
from jax.experimental.pallas import tpu as pltpu, tpu_sc as plsc

N = 10000
E = 320000
D = 128
DH = D // 2
NP = 10112
NR = NP // 128
NC = 2
NS = 16
NW = NC * NS
EP = 327680
EPW = EP // NW
CHUNK = 128
CH = EPW // CHUNK
EPT = EP // NS
CHT = EPT // CHUNK
UNROLL = 8
NPH = CHT // UNROLL
RPT = NP // NS

_mesh = plsc.VectorSubcoreMesh(core_axis_name="c", subcore_axis_name="s")


DPH = 8


@functools.partial(
    pl.kernel,
    mesh=_mesh,
    out_type=jax.ShapeDtypeStruct((NC * 2 * NP,), jnp.float32),
    scratch_types=[
        pltpu.VMEM((CH, CHUNK), jnp.int32),
        pltpu.VMEM((CH, CHUNK), jnp.int32),
        pltpu.VMEM((CHUNK,), jnp.float32),
        pltpu.VMEM((NP,), jnp.float32),
        pltpu.VMEM_SHARED((NP,), jnp.float32),
        pltpu.VMEM_SHARED((NP,), jnp.float32),
        pltpu.SemaphoreType.DMA,
    ],
    compiler_params=pltpu.CompilerParams(use_tc_tiling_on_sc=False),
)
def _deg_kernel(src_h, dst_h, out_h, sidx, didx, ones_v, stage_v,
                dout_sh, din_sh, dsem):
    cid = lax.axis_index("c")
    sid = lax.axis_index("s")

    def _fill_ones(i, _):
        ones_v[pl.ds(i * 16, 16)] = jnp.ones((16,), jnp.float32)
        return 0

    lax.fori_loop(0, CHUNK // 16, _fill_ones, 0)

    def _fill_zero(i, _):
        stage_v[pl.ds(i * 16, 16)] = jnp.zeros((16,), jnp.float32)
        return 0

    lax.fori_loop(0, NP // 16, _fill_zero, 0)

    pltpu.sync_copy(src_h.at[sid, pl.ds(cid * CH, CH)], sidx)
    pltpu.sync_copy(dst_h.at[sid, pl.ds(cid * CH, CH)], didx)

    @pl.when(sid == 0)
    def _():
        pltpu.sync_copy(stage_v, dout_sh)

    @pl.when(sid == 1)
    def _():
        pltpu.sync_copy(stage_v, din_sh)

    plsc.subcore_barrier()

    def _burst(p, _):
        descs = []
        for q in range(DPH):
            j = p * DPH + q
            descs.append(pltpu.async_copy(
                ones_v, dout_sh.at[sidx.at[j]], dsem, add=True))
            descs.append(pltpu.async_copy(
                ones_v, din_sh.at[didx.at[j]], dsem, add=True))
        for d in descs:
            d.wait()
        return 0

    lax.fori_loop(0, CH // DPH, _burst, 0)

    plsc.subcore_barrier()

    @pl.when(sid == 0)
    def _():
        pltpu.sync_copy(dout_sh, stage_v)
        pltpu.sync_copy(stage_v, out_h.at[pl.ds(cid * 2 * NP, NP)])

    @pl.when(sid == 1)
    def _():
        pltpu.sync_copy(din_sh, stage_v)
        pltpu.sync_copy(stage_v, out_h.at[pl.ds(cid * 2 * NP + NP, NP)])


def _mm_body(x_ref, w_ref, h_ref):
    h_ref[...] = jnp.dot(x_ref[...], w_ref[...],
                         preferred_element_type=jnp.float32)


def _matmul(x, W):
    return pl.pallas_call(
        _mm_body,
        out_shape=jax.ShapeDtypeStruct((N, D), jnp.float32),
    )(x, W)


def _scale_body(h_ref, dp_ref, g_ref, nd_ref):
    deg_out = dp_ref[0, :] + dp_ref[2, :]
    deg_in = dp_ref[1, :] + dp_ref[3, :]
    ns = jnp.where(deg_out > 0, 1.0 / jnp.sqrt(jnp.maximum(deg_out, 1.0)), 0.0)
    nd_ref[...] = jnp.where(
        deg_in > 0, 1.0 / jnp.sqrt(jnp.maximum(deg_in, 1.0)), 0.0)
    hs = h_ref[...] * ns[:N, None]
    g_ref[0, pl.ds(0, N)] = hs[:, :DH]
    g_ref[1, pl.ds(0, N)] = hs[:, DH:]
    pad = jnp.zeros((NP - N, DH), jnp.float32)
    g_ref[0, pl.ds(N, NP - N)] = pad
    g_ref[1, pl.ds(N, NP - N)] = pad


def _scale(h, degp2):
    return pl.pallas_call(
        _scale_body,
        out_shape=[
            jax.ShapeDtypeStruct((NC, NP, DH), jnp.float32),
            jax.ShapeDtypeStruct((NP,), jnp.float32),
        ],
    )(h, degp2)


@functools.partial(
    pl.kernel,
    mesh=_mesh,
    out_type=jax.ShapeDtypeStruct((NC, NP, DH), jnp.float32),
    scratch_types=[
        pltpu.VMEM((UNROLL, CHUNK), jnp.int32),
        pltpu.VMEM((UNROLL, CHUNK), jnp.int32),
        pltpu.VMEM((UNROLL, CHUNK), jnp.int32),
        pltpu.VMEM((UNROLL, CHUNK), jnp.int32),
        pltpu.VMEM((CHUNK, DH), jnp.float32),
        pltpu.VMEM((CHUNK, DH), jnp.float32),
        pltpu.VMEM((CHUNK, DH), jnp.float32),
        pltpu.VMEM((CHUNK, DH), jnp.float32),
        pltpu.VMEM_SHARED((NP, DH), jnp.float32),
        pltpu.VMEM_SHARED((NP, DH), jnp.float32),
        pltpu.SemaphoreType.DMA,
        pltpu.SemaphoreType.DMA,
        pltpu.SemaphoreType.DMA,
        pltpu.SemaphoreType.DMA,
        pltpu.SemaphoreType.DMA,
        pltpu.SemaphoreType.DMA,
        pltpu.SemaphoreType.DMA,
        pltpu.SemaphoreType.DMA,
        pltpu.SemaphoreType.DMA,
    ],
    compiler_params=pltpu.CompilerParams(use_tc_tiling_on_sc=False),
)
def _agg_kernel(g_h, src_h, dst_h, out_h, sidxa, didxa, sidxb, didxb,
                buf0, buf1, buf2, buf3, g_sh, agg_sh,
                gsem0, gsem1, gsem2, gsem3,
                ssem0, ssem1, ssem2, ssem3, stsem):
    cid = lax.axis_index("c")
    sid = lax.axis_index("s")
    row0 = sid * RPT
    _tail = RPT % CHUNK

    for k in range(RPT // CHUNK):
        pltpu.sync_copy(g_h.at[cid, pl.ds(row0 + k * CHUNK, CHUNK)], buf0)
        pltpu.sync_copy(buf0, g_sh.at[pl.ds(row0 + k * CHUNK, CHUNK)])
    if _tail:
        _s0 = row0 + (RPT // CHUNK) * CHUNK
        pltpu.sync_copy(g_h.at[cid, pl.ds(_s0, _tail)], buf0.at[pl.ds(0, _tail)])
        pltpu.sync_copy(buf0.at[pl.ds(0, _tail)], g_sh.at[pl.ds(_s0, _tail)])

    def _zrow(r, _):
        for cc in range(DH // 16):
            buf0[r, pl.ds(cc * 16, 16)] = jnp.zeros((16,), jnp.float32)
        return 0

    lax.fori_loop(0, CHUNK, _zrow, 0)

    for k in range(RPT // CHUNK):
        pltpu.sync_copy(buf0, agg_sh.at[pl.ds(row0 + k * CHUNK, CHUNK)])
    if _tail:
        pltpu.sync_copy(
            buf0.at[pl.ds(0, _tail)],
            agg_sh.at[pl.ds(row0 + (RPT // CHUNK) * CHUNK, _tail)],
        )

    plsc.subcore_barrier()

    bufs = (buf0, buf1, buf2, buf3)
    gsems = (gsem0, gsem1, gsem2, gsem3)
    ssems = (ssem0, ssem1, ssem2, ssem3)
    NB = 4

    def _run_phase(sidx, didx):
        gd = [pltpu.async_copy(g_sh.at[sidx.at[q]], bufs[q], gsems[q])
              for q in range(NB - 1)]
        sd = [None] * UNROLL
        for j in range(UNROLL):
            if j + NB - 1 < UNROLL:
                if j >= 1:
                    sd[j - 1].wait()
                gd.append(pltpu.async_copy(
                    g_sh.at[sidx.at[j + NB - 1]], bufs[(j + NB - 1) % NB],
                    gsems[(j + NB - 1) % NB]))
            gd[j].wait()
            sd[j] = pltpu.async_copy(
                bufs[j % NB], agg_sh.at[didx.at[j]], ssems[j % NB], add=True)
        for j in range(UNROLL - NB, UNROLL):
            sd[j].wait()

    pltpu.sync_copy(src_h.at[sid, pl.ds(0, UNROLL)], sidxa)
    pltpu.sync_copy(dst_h.at[sid, pl.ds(0, UNROLL)], didxa)

    def _phase_pair(pp, _):
        p = pp * 2
        b1 = (p + 1) * UNROLL
        s1 = pltpu.async_copy(src_h.at[sid, pl.ds(b1, UNROLL)], sidxb, stsem)
        s2 = pltpu.async_copy(dst_h.at[sid, pl.ds(b1, UNROLL)], didxb, stsem)
        _run_phase(sidxa, didxa)
        s1.wait()
        s2.wait()
        b2 = jnp.minimum((p + 2) * UNROLL, CHT - UNROLL)
        s3 = pltpu.async_copy(src_h.at[sid, pl.ds(b2, UNROLL)], sidxa, stsem)
        s4 = pltpu.async_copy(dst_h.at[sid, pl.ds(b2, UNROLL)], didxa, stsem)
        _run_phase(sidxb, didxb)
        s3.wait()
        s4.wait()
        return 0

    lax.fori_loop(0, NPH // 2, _phase_pair, 0)

    plsc.subcore_barrier()

    for k in range(RPT // CHUNK):
        pltpu.sync_copy(agg_sh.at[pl.ds(row0 + k * CHUNK, CHUNK)], buf0)
        pltpu.sync_copy(buf0, out_h.at[cid, pl.ds(row0 + k * CHUNK, CHUNK)])
    if _tail:
        _t0 = row0 + (RPT // CHUNK) * CHUNK
        pltpu.sync_copy(agg_sh.at[pl.ds(_t0, _tail)], buf0.at[pl.ds(0, _tail)])
        pltpu.sync_copy(buf0.at[pl.ds(0, _tail)], out_h.at[cid, pl.ds(_t0, _tail)])


def _final_body(agg_ref, nd_ref, b_ref, out_ref):
    full = jnp.concatenate(
        [agg_ref[0, pl.ds(0, N)], agg_ref[1, pl.ds(0, N)]], axis=1)
    nd = nd_ref[pl.ds(0, N)]
    out_ref[...] = full * nd[:, None] + b_ref[...][None, :]


def _final(agg, nd, b):
    return pl.pallas_call(
        _final_body,
        out_shape=jax.ShapeDtypeStruct((N, D), jnp.float32),
    )(agg, nd, b)


def kernel(x, edge_index, W, b):
    src = edge_index[0].astype(jnp.int32)
    dst = edge_index[1].astype(jnp.int32)
    pad = jnp.full((EP - E,), N, jnp.int32)
    srct = jnp.concatenate([src, pad]).reshape(NS, CHT, CHUNK)
    dstt = jnp.concatenate([dst, pad]).reshape(NS, CHT, CHUNK)

    degp = _deg_kernel(srct, dstt)
    h = _matmul(x, W)
    g2, nd = _scale(h, degp.reshape(NC * 2, NP))

    agg = _agg_kernel(g2, srct, dstt)
    return _final(agg, nd, b)


# speedup vs baseline: 1.0335x; 1.0335x over previous
import functools

import jax
import jax.numpy as jnp
from jax import lax
from jax.experimental import pallas as pl
from jax.experimental.pallas import tpu as pltpu, tpu_sc as plsc

N = 10000
E = 320000
D = 128
DH = D // 2
NP = 10112
NR = NP // 128
NC = 2
NS = 16
NW = NC * NS
EP = 327680
EPW = EP // NW
CHUNK = 128
CH = EPW // CHUNK
EPT = EP // NS
CHT = EPT // CHUNK
UNROLL = 8
NPH = CHT // UNROLL
RPT = NP // NS

_mesh = plsc.VectorSubcoreMesh(core_axis_name="c", subcore_axis_name="s")


DPH = 8


@functools.partial(
    pl.kernel,
    mesh=_mesh,
    out_type=jax.ShapeDtypeStruct((NC * 2 * NP,), jnp.float32),
    scratch_types=[
        pltpu.VMEM((CH, CHUNK), jnp.int32),
        pltpu.VMEM((CH, CHUNK), jnp.int32),
        pltpu.VMEM((CHUNK,), jnp.float32),
        pltpu.VMEM((NP,), jnp.float32),
        pltpu.VMEM_SHARED((NP,), jnp.float32),
        pltpu.VMEM_SHARED((NP,), jnp.float32),
        pltpu.SemaphoreType.DMA,
    ],
    compiler_params=pltpu.CompilerParams(use_tc_tiling_on_sc=False),
)
def _deg_kernel(src_h, dst_h, out_h, sidx, didx, ones_v, stage_v,
                dout_sh, din_sh, dsem):
    cid = lax.axis_index("c")
    sid = lax.axis_index("s")

    def _fill_ones(i, _):
        ones_v[pl.ds(i * 16, 16)] = jnp.ones((16,), jnp.float32)
        return 0

    lax.fori_loop(0, CHUNK // 16, _fill_ones, 0)

    def _fill_zero(i, _):
        stage_v[pl.ds(i * 16, 16)] = jnp.zeros((16,), jnp.float32)
        return 0

    lax.fori_loop(0, NP // 16, _fill_zero, 0)

    pltpu.sync_copy(src_h.at[sid, pl.ds(cid * CH, CH)], sidx)
    pltpu.sync_copy(dst_h.at[sid, pl.ds(cid * CH, CH)], didx)

    @pl.when(sid == 0)
    def _():
        pltpu.sync_copy(stage_v, dout_sh)

    @pl.when(sid == 1)
    def _():
        pltpu.sync_copy(stage_v, din_sh)

    plsc.subcore_barrier()

    def _burst(p, _):
        descs = []
        for q in range(DPH):
            j = p * DPH + q
            descs.append(pltpu.async_copy(
                ones_v, dout_sh.at[sidx.at[j]], dsem, add=True))
            descs.append(pltpu.async_copy(
                ones_v, din_sh.at[didx.at[j]], dsem, add=True))
        for d in descs:
            d.wait()
        return 0

    lax.fori_loop(0, CH // DPH, _burst, 0)

    plsc.subcore_barrier()

    @pl.when(sid == 0)
    def _():
        pltpu.sync_copy(dout_sh, stage_v)
        pltpu.sync_copy(stage_v, out_h.at[pl.ds(cid * 2 * NP, NP)])

    @pl.when(sid == 1)
    def _():
        pltpu.sync_copy(din_sh, stage_v)
        pltpu.sync_copy(stage_v, out_h.at[pl.ds(cid * 2 * NP + NP, NP)])


def _dense_body(x_ref, w_ref, dp_ref, g_ref, nd_ref):
    deg_out = dp_ref[0, :] + dp_ref[2, :]
    deg_in = dp_ref[1, :] + dp_ref[3, :]
    ns = jnp.where(deg_out > 0, 1.0 / jnp.sqrt(jnp.maximum(deg_out, 1.0)), 0.0)
    nd_ref[...] = jnp.where(
        deg_in > 0, 1.0 / jnp.sqrt(jnp.maximum(deg_in, 1.0)), 0.0)
    h = jnp.dot(x_ref[...], w_ref[...], preferred_element_type=jnp.float32)
    hs = h * ns[:N, None]
    g_ref[0, pl.ds(0, N)] = hs[:, :DH]
    g_ref[1, pl.ds(0, N)] = hs[:, DH:]
    pad = jnp.zeros((NP - N, DH), jnp.float32)
    g_ref[0, pl.ds(N, NP - N)] = pad
    g_ref[1, pl.ds(N, NP - N)] = pad


def _dense(x, W, degp2):
    return pl.pallas_call(
        _dense_body,
        out_shape=[
            jax.ShapeDtypeStruct((NC, NP, DH), jnp.float32),
            jax.ShapeDtypeStruct((NP,), jnp.float32),
        ],
    )(x, W, degp2)


@functools.partial(
    pl.kernel,
    mesh=_mesh,
    out_type=jax.ShapeDtypeStruct((NC, NP, DH), jnp.float32),
    scratch_types=[
        pltpu.VMEM((UNROLL, CHUNK), jnp.int32),
        pltpu.VMEM((UNROLL, CHUNK), jnp.int32),
        pltpu.VMEM((UNROLL, CHUNK), jnp.int32),
        pltpu.VMEM((UNROLL, CHUNK), jnp.int32),
        pltpu.VMEM((CHUNK, DH), jnp.float32),
        pltpu.VMEM((CHUNK, DH), jnp.float32),
        pltpu.VMEM_SHARED((NP, DH), jnp.float32),
        pltpu.VMEM_SHARED((NP, DH), jnp.float32),
        pltpu.SemaphoreType.DMA,
        pltpu.SemaphoreType.DMA,
        pltpu.SemaphoreType.DMA,
        pltpu.SemaphoreType.DMA,
        pltpu.SemaphoreType.DMA,
    ],
    compiler_params=pltpu.CompilerParams(use_tc_tiling_on_sc=False),
)
def _agg_kernel(g_h, src_h, dst_h, out_h, sidxa, didxa, sidxb, didxb,
                buf0, buf1, g_sh, agg_sh,
                gsem0, gsem1, ssem0, ssem1, stsem):
    cid = lax.axis_index("c")
    sid = lax.axis_index("s")
    row0 = sid * RPT
    _tail = RPT % CHUNK

    for k in range(RPT // CHUNK):
        pltpu.sync_copy(g_h.at[cid, pl.ds(row0 + k * CHUNK, CHUNK)], buf0)
        pltpu.sync_copy(buf0, g_sh.at[pl.ds(row0 + k * CHUNK, CHUNK)])
    if _tail:
        _s0 = row0 + (RPT // CHUNK) * CHUNK
        pltpu.sync_copy(g_h.at[cid, pl.ds(_s0, _tail)], buf0.at[pl.ds(0, _tail)])
        pltpu.sync_copy(buf0.at[pl.ds(0, _tail)], g_sh.at[pl.ds(_s0, _tail)])

    def _zrow(r, _):
        for cc in range(DH // 16):
            buf0[r, pl.ds(cc * 16, 16)] = jnp.zeros((16,), jnp.float32)
        return 0

    lax.fori_loop(0, CHUNK, _zrow, 0)

    for k in range(RPT // CHUNK):
        pltpu.sync_copy(buf0, agg_sh.at[pl.ds(row0 + k * CHUNK, CHUNK)])
    if _tail:
        pltpu.sync_copy(
            buf0.at[pl.ds(0, _tail)],
            agg_sh.at[pl.ds(row0 + (RPT // CHUNK) * CHUNK, _tail)],
        )

    plsc.subcore_barrier()

    bufs = (buf0, buf1)
    gsems = (gsem0, gsem1)
    ssems = (ssem0, ssem1)
    NB = 2

    def _run_phase(sidx, didx):
        gd = [pltpu.async_copy(g_sh.at[sidx.at[q]], bufs[q], gsems[q])
              for q in range(NB - 1)]
        sd = [None] * UNROLL
        for j in range(UNROLL):
            if j + NB - 1 < UNROLL:
                if j >= 1:
                    sd[j - 1].wait()
                gd.append(pltpu.async_copy(
                    g_sh.at[sidx.at[j + NB - 1]], bufs[(j + NB - 1) % NB],
                    gsems[(j + NB - 1) % NB]))
            gd[j].wait()
            sd[j] = pltpu.async_copy(
                bufs[j % NB], agg_sh.at[didx.at[j]], ssems[j % NB], add=True)
        for j in range(UNROLL - NB, UNROLL):
            sd[j].wait()

    pltpu.sync_copy(src_h.at[sid, pl.ds(0, UNROLL)], sidxa)
    pltpu.sync_copy(dst_h.at[sid, pl.ds(0, UNROLL)], didxa)

    def _phase_pair(pp, _):
        p = pp * 2
        b1 = (p + 1) * UNROLL
        s1 = pltpu.async_copy(src_h.at[sid, pl.ds(b1, UNROLL)], sidxb, stsem)
        s2 = pltpu.async_copy(dst_h.at[sid, pl.ds(b1, UNROLL)], didxb, stsem)
        _run_phase(sidxa, didxa)
        s1.wait()
        s2.wait()
        b2 = jnp.minimum((p + 2) * UNROLL, CHT - UNROLL)
        s3 = pltpu.async_copy(src_h.at[sid, pl.ds(b2, UNROLL)], sidxa, stsem)
        s4 = pltpu.async_copy(dst_h.at[sid, pl.ds(b2, UNROLL)], didxa, stsem)
        _run_phase(sidxb, didxb)
        s3.wait()
        s4.wait()
        return 0

    lax.fori_loop(0, NPH // 2, _phase_pair, 0)

    plsc.subcore_barrier()

    for k in range(RPT // CHUNK):
        pltpu.sync_copy(agg_sh.at[pl.ds(row0 + k * CHUNK, CHUNK)], buf0)
        pltpu.sync_copy(buf0, out_h.at[cid, pl.ds(row0 + k * CHUNK, CHUNK)])
    if _tail:
        _t0 = row0 + (RPT // CHUNK) * CHUNK
        pltpu.sync_copy(agg_sh.at[pl.ds(_t0, _tail)], buf0.at[pl.ds(0, _tail)])
        pltpu.sync_copy(buf0.at[pl.ds(0, _tail)], out_h.at[cid, pl.ds(_t0, _tail)])


def _final_body(agg_ref, nd_ref, b_ref, out_ref):
    full = jnp.concatenate(
        [agg_ref[0, pl.ds(0, N)], agg_ref[1, pl.ds(0, N)]], axis=1)
    nd = nd_ref[pl.ds(0, N)]
    out_ref[...] = full * nd[:, None] + b_ref[...][None, :]


def _final(agg, nd, b):
    return pl.pallas_call(
        _final_body,
        out_shape=jax.ShapeDtypeStruct((N, D), jnp.float32),
    )(agg, nd, b)


def kernel(x, edge_index, W, b):
    src = edge_index[0].astype(jnp.int32)
    dst = edge_index[1].astype(jnp.int32)
    pad = jnp.full((EP - E,), N, jnp.int32)
    srct = jnp.concatenate([src, pad]).reshape(NS, CHT, CHUNK)
    dstt = jnp.concatenate([dst, pad]).reshape(NS, CHT, CHUNK)

    degp = _deg_kernel(srct, dstt)
    g2, nd = _dense(x, W, degp.reshape(NC * 2, NP))

    agg = _agg_kernel(g2, srct, dstt)
    return _final(agg, nd, b)


# speedup vs baseline: 1.0460x; 1.0121x over previous
import functools

import jax
import jax.numpy as jnp
from jax import lax
from jax.experimental import pallas as pl
from jax.experimental.pallas import tpu as pltpu, tpu_sc as plsc

N = 10000
E = 320000
D = 128
DH = D // 2
NP = 10112
NR = NP // 128
NC = 2
NS = 16
NW = NC * NS
EP = 327680
EPW = EP // NW
CHUNK = 128
CH = EPW // CHUNK
EPT = EP // NS
CHT = EPT // CHUNK
UNROLL = 8
NPH = CHT // UNROLL
RPT = NP // NS

_mesh = plsc.VectorSubcoreMesh(core_axis_name="c", subcore_axis_name="s")


DPH = 8


@functools.partial(
    pl.kernel,
    mesh=_mesh,
    out_type=jax.ShapeDtypeStruct((NC * 2 * NP,), jnp.float32),
    scratch_types=[
        pltpu.VMEM((CH, CHUNK), jnp.int32),
        pltpu.VMEM((CH, CHUNK), jnp.int32),
        pltpu.VMEM((CHUNK,), jnp.float32),
        pltpu.VMEM((NP,), jnp.float32),
        pltpu.VMEM_SHARED((NP,), jnp.float32),
        pltpu.VMEM_SHARED((NP,), jnp.float32),
        pltpu.SemaphoreType.DMA,
    ],
    compiler_params=pltpu.CompilerParams(use_tc_tiling_on_sc=False),
)
def _deg_kernel(src_h, dst_h, out_h, sidx, didx, ones_v, stage_v,
                dout_sh, din_sh, dsem):
    cid = lax.axis_index("c")
    sid = lax.axis_index("s")

    def _fill_ones(i, _):
        ones_v[pl.ds(i * 16, 16)] = jnp.ones((16,), jnp.float32)
        return 0

    lax.fori_loop(0, CHUNK // 16, _fill_ones, 0)

    def _fill_zero(i, _):
        stage_v[pl.ds(i * 16, 16)] = jnp.zeros((16,), jnp.float32)
        return 0

    lax.fori_loop(0, NP // 16, _fill_zero, 0)

    pltpu.sync_copy(src_h.at[sid, pl.ds(cid * CH, CH)], sidx)
    pltpu.sync_copy(dst_h.at[sid, pl.ds(cid * CH, CH)], didx)

    @pl.when(sid == 0)
    def _():
        pltpu.sync_copy(stage_v, dout_sh)

    @pl.when(sid == 1)
    def _():
        pltpu.sync_copy(stage_v, din_sh)

    plsc.subcore_barrier()

    def _burst(p, _):
        descs = []
        for q in range(DPH):
            j = p * DPH + q
            descs.append(pltpu.async_copy(
                ones_v, dout_sh.at[sidx.at[j]], dsem, add=True))
            descs.append(pltpu.async_copy(
                ones_v, din_sh.at[didx.at[j]], dsem, add=True))
        for d in descs:
            d.wait()
        return 0

    lax.fori_loop(0, CH // DPH, _burst, 0)

    plsc.subcore_barrier()

    @pl.when(sid == 0)
    def _():
        pltpu.sync_copy(dout_sh, stage_v)
        pltpu.sync_copy(stage_v, out_h.at[pl.ds(cid * 2 * NP, NP)])

    @pl.when(sid == 1)
    def _():
        pltpu.sync_copy(din_sh, stage_v)
        pltpu.sync_copy(stage_v, out_h.at[pl.ds(cid * 2 * NP + NP, NP)])


def _dense_body(x_ref, w_ref, dp_ref, g_ref, nd_ref):
    deg_out = dp_ref[0, :] + dp_ref[2, :]
    deg_in = dp_ref[1, :] + dp_ref[3, :]
    ns = jnp.where(deg_out > 0, 1.0 / jnp.sqrt(jnp.maximum(deg_out, 1.0)), 0.0)
    nd_ref[...] = jnp.where(
        deg_in > 0, 1.0 / jnp.sqrt(jnp.maximum(deg_in, 1.0)), 0.0)
    h = jnp.dot(x_ref[...], w_ref[...], preferred_element_type=jnp.float32)
    hs = h * ns[:N, None]
    g_ref[0, pl.ds(0, N)] = hs[:, :DH]
    g_ref[1, pl.ds(0, N)] = hs[:, DH:]
    pad = jnp.zeros((NP - N, DH), jnp.float32)
    g_ref[0, pl.ds(N, NP - N)] = pad
    g_ref[1, pl.ds(N, NP - N)] = pad


def _dense(x, W, degp2):
    return pl.pallas_call(
        _dense_body,
        out_shape=[
            jax.ShapeDtypeStruct((NC, NP, DH), jnp.float32),
            jax.ShapeDtypeStruct((NP,), jnp.float32),
        ],
    )(x, W, degp2)


@functools.partial(
    pl.kernel,
    mesh=_mesh,
    out_type=jax.ShapeDtypeStruct((NC, NP, DH), jnp.float32),
    scratch_types=[
        pltpu.VMEM((UNROLL, CHUNK), jnp.int32),
        pltpu.VMEM((UNROLL, CHUNK), jnp.int32),
        pltpu.VMEM((UNROLL, CHUNK), jnp.int32),
        pltpu.VMEM((UNROLL, CHUNK), jnp.int32),
        pltpu.VMEM((CHUNK, DH), jnp.float32),
        pltpu.VMEM((CHUNK, DH), jnp.float32),
        pltpu.VMEM_SHARED((NP, DH), jnp.float32),
        pltpu.VMEM_SHARED((NP, DH), jnp.float32),
        pltpu.SemaphoreType.DMA,
        pltpu.SemaphoreType.DMA,
        pltpu.SemaphoreType.DMA,
        pltpu.SemaphoreType.DMA,
        pltpu.SemaphoreType.DMA,
    ],
    compiler_params=pltpu.CompilerParams(use_tc_tiling_on_sc=False),
)
def _agg_kernel(g_h, src_h, dst_h, out_h, sidxa, didxa, sidxb, didxb,
                buf0, buf1, g_sh, agg_sh,
                gsem0, gsem1, ssem0, ssem1, stsem):
    cid = lax.axis_index("c")
    sid = lax.axis_index("s")
    row0 = sid * RPT
    _tail = RPT % CHUNK

    _sizes = [CHUNK] * (RPT // CHUNK) + ([_tail] if _tail else [])

    def _bslice(b, sz):
        return b if sz == CHUNK else b.at[pl.ds(0, sz)]

    _bufs2 = (buf0, buf1)
    _isems = (gsem0, gsem1)
    _osems = (ssem0, ssem1)

    d_in = pltpu.async_copy(
        g_h.at[cid, pl.ds(row0, _sizes[0])], _bslice(buf0, _sizes[0]), gsem0)
    prev_out = [None, None]
    _off = 0
    for k, sz in enumerate(_sizes):
        if k + 1 < len(_sizes):
            nb = (k + 1) % 2
            nsz = _sizes[k + 1]
            if prev_out[nb] is not None:
                prev_out[nb].wait()
            d_next = pltpu.async_copy(
                g_h.at[cid, pl.ds(row0 + _off + sz, nsz)],
                _bslice(_bufs2[nb], nsz), _isems[nb])
        d_in.wait()
        prev_out[k % 2] = pltpu.async_copy(
            _bslice(_bufs2[k % 2], sz),
            g_sh.at[pl.ds(row0 + _off, sz)], _osems[k % 2])
        _off += sz
        if k + 1 < len(_sizes):
            d_in = d_next
    for po in prev_out:
        if po is not None:
            po.wait()

    def _zrow(r, _):
        for cc in range(DH // 16):
            buf0[r, pl.ds(cc * 16, 16)] = jnp.zeros((16,), jnp.float32)
        return 0

    lax.fori_loop(0, CHUNK, _zrow, 0)

    _zd = []
    _off = 0
    for sz in _sizes:
        _zd.append(pltpu.async_copy(
            _bslice(buf0, sz), agg_sh.at[pl.ds(row0 + _off, sz)], ssem0))
        _off += sz
    for d in _zd:
        d.wait()

    plsc.subcore_barrier()

    bufs = (buf0, buf1)
    gsems = (gsem0, gsem1)
    ssems = (ssem0, ssem1)
    NB = 2

    def _run_phase(sidx, didx):
        gd = [pltpu.async_copy(g_sh.at[sidx.at[q]], bufs[q], gsems[q])
              for q in range(NB - 1)]
        sd = [None] * UNROLL
        for j in range(UNROLL):
            if j + NB - 1 < UNROLL:
                if j >= 1:
                    sd[j - 1].wait()
                gd.append(pltpu.async_copy(
                    g_sh.at[sidx.at[j + NB - 1]], bufs[(j + NB - 1) % NB],
                    gsems[(j + NB - 1) % NB]))
            gd[j].wait()
            sd[j] = pltpu.async_copy(
                bufs[j % NB], agg_sh.at[didx.at[j]], ssems[j % NB], add=True)
        for j in range(UNROLL - NB, UNROLL):
            sd[j].wait()

    pltpu.sync_copy(src_h.at[sid, pl.ds(0, UNROLL)], sidxa)
    pltpu.sync_copy(dst_h.at[sid, pl.ds(0, UNROLL)], didxa)

    def _phase_pair(pp, _):
        p = pp * 2
        b1 = (p + 1) * UNROLL
        s1 = pltpu.async_copy(src_h.at[sid, pl.ds(b1, UNROLL)], sidxb, stsem)
        s2 = pltpu.async_copy(dst_h.at[sid, pl.ds(b1, UNROLL)], didxb, stsem)
        _run_phase(sidxa, didxa)
        s1.wait()
        s2.wait()
        b2 = jnp.minimum((p + 2) * UNROLL, CHT - UNROLL)
        s3 = pltpu.async_copy(src_h.at[sid, pl.ds(b2, UNROLL)], sidxa, stsem)
        s4 = pltpu.async_copy(dst_h.at[sid, pl.ds(b2, UNROLL)], didxa, stsem)
        _run_phase(sidxb, didxb)
        s3.wait()
        s4.wait()
        return 0

    lax.fori_loop(0, NPH // 2, _phase_pair, 0)

    plsc.subcore_barrier()

    d_in = pltpu.async_copy(
        agg_sh.at[pl.ds(row0, _sizes[0])], _bslice(buf0, _sizes[0]), gsem0)
    prev_out = [None, None]
    _off = 0
    for k, sz in enumerate(_sizes):
        if k + 1 < len(_sizes):
            nb = (k + 1) % 2
            nsz = _sizes[k + 1]
            if prev_out[nb] is not None:
                prev_out[nb].wait()
            d_next = pltpu.async_copy(
                agg_sh.at[pl.ds(row0 + _off + sz, nsz)],
                _bslice(_bufs2[nb], nsz), _isems[nb])
        d_in.wait()
        prev_out[k % 2] = pltpu.async_copy(
            _bslice(_bufs2[k % 2], sz),
            out_h.at[cid, pl.ds(row0 + _off, sz)], _osems[k % 2])
        _off += sz
        if k + 1 < len(_sizes):
            d_in = d_next
    for po in prev_out:
        if po is not None:
            po.wait()


def _final_body(agg_ref, nd_ref, b_ref, out_ref):
    full = jnp.concatenate(
        [agg_ref[0, pl.ds(0, N)], agg_ref[1, pl.ds(0, N)]], axis=1)
    nd = nd_ref[pl.ds(0, N)]
    out_ref[...] = full * nd[:, None] + b_ref[...][None, :]


def _final(agg, nd, b):
    return pl.pallas_call(
        _final_body,
        out_shape=jax.ShapeDtypeStruct((N, D), jnp.float32),
    )(agg, nd, b)


def kernel(x, edge_index, W, b):
    src = edge_index[0].astype(jnp.int32)
    dst = edge_index[1].astype(jnp.int32)
    pad = jnp.full((EP - E,), N, jnp.int32)
    srct = jnp.concatenate([src, pad]).reshape(NS, CHT, CHUNK)
    dstt = jnp.concatenate([dst, pad]).reshape(NS, CHT, CHUNK)

    degp = _deg_kernel(srct, dstt)
    g2, nd = _dense(x, W, degp.reshape(NC * 2, NP))

    agg = _agg_kernel(g2, srct, dstt)
    return _final(agg, nd, b)
